# baseline (speedup 1.0000x reference)
import functools

import jax
import jax.numpy as jnp
from jax import lax
from jax.experimental import pallas as pl
from jax.experimental.pallas import tpu as pltpu
from jax.experimental.pallas import tpu_sc as plsc

N = 10000
E = 320000
D = 128
NS = 16
NC = 2
CH = 40
EPT = E // (NC * NS)
NCH = EPT // CH
RPT = N // NS
XRT = 632
INV_BN = float((1.0 + 1e-5) ** -0.5)
EPS = 1e-6

_f32 = jnp.float32


def _sigmoid16(x):
    return 1.0 / (1.0 + jnp.exp(-x))


def _zero_acc(buf, acc, base8):
    def zrow(r, carry):
        for g in range(D // 16):
            buf[r, pl.ds(g * 16, 16)] = jnp.zeros((16,), _f32)
        return carry
    lax.fori_loop(0, CH, zrow, 0)
    for q in range(XRT // CH):
        pltpu.sync_copy(buf, acc.at[pl.ds(base8 + q * CH, CH)])
    rem = XRT - (XRT // CH) * CH
    pltpu.sync_copy(buf.at[pl.ds(0, rem)],
                    acc.at[pl.ds(base8 + (XRT // CH) * CH, rem)])


def _copy_idx(src_ref, dst_ref):
    for off in (0, 16, CH - 16):
        s = pl.ds(off, 16)
        dst_ref[s] = src_ref[s]


@functools.partial(jax.jit, static_argnames=("has_b3",))
def _sc_hat_sig(src, dst, b1, b2, b3e, *, has_b3):
    out_type = (
        jax.ShapeDtypeStruct((E, D), _f32),
        jax.ShapeDtypeStruct((NC, N, D), _f32),
    )
    scratch = (
        [pltpu.VMEM((CH,), jnp.int32)] * 6
        + [pltpu.VMEM((CH, D), _f32)] * 8
        + [pltpu.VMEM_SHARED((N, D), _f32)]
        + [pltpu.SemaphoreType.DMA] * 6
    )

    def body(*refs):
        it = iter(refs)
        src_h = next(it); dst_h = next(it); b1_h = next(it); b2_h = next(it)
        b3e_h = next(it) if has_b3 else None
        hat_h = next(it); acc_h = next(it)
        srcv = [next(it) for _ in range(2)]
        dstv = [next(it) for _ in range(2)]
        dsts = [next(it) for _ in range(2)]
        g1 = [next(it) for _ in range(2)]
        g2 = [next(it) for _ in range(2)]
        hb = [next(it) for _ in range(2)]
        sg = [next(it) for _ in range(2)]
        acc = next(it)
        semg = [next(it) for _ in range(2)]
        semw = [next(it) for _ in range(2)]
        sems = [next(it) for _ in range(2)]

        cid = lax.axis_index("c")
        sid = lax.axis_index("s")
        base8 = (sid * RPT) // 8 * 8
        tbase = (cid * NS + sid) * EPT

        _zero_acc(g1[0], acc, base8)
        plsc.subcore_barrier()

        def issue(c, b):
            base = tbase + c * CH
            pltpu.sync_copy(src_h.at[pl.ds(base, CH)], srcv[b])
            pltpu.sync_copy(dst_h.at[pl.ds(base, CH)], dstv[b])
            pltpu.async_copy(b1_h.at[srcv[b]], g1[b], semg[b])
            pltpu.async_copy(b2_h.at[dstv[b]], g2[b], semg[b])
            if has_b3:
                pltpu.async_copy(b3e_h.at[pl.ds(base, CH)], hb[b], semg[b])

        issue(0, 0)

        def outer(cc, carry):
            for b in (0, 1):
                ob = 1 - b
                c = cc * 2 + b
                @pl.when(c >= 1)
                def _():
                    pltpu.make_async_copy(
                        hb[ob], hat_h.at[pl.ds(tbase + (c - 1) * CH, CH)],
                        semw[ob]).wait()
                @pl.when(c + 1 < NCH)
                def _():
                    issue(c + 1, ob)
                pltpu.make_async_copy(
                    b1_h.at[srcv[b]], g1[b], semg[b]).wait()
                pltpu.make_async_copy(
                    b2_h.at[dstv[b]], g2[b], semg[b]).wait()
                if has_b3:
                    pltpu.make_async_copy(
                        b3e_h.at[pl.ds(tbase + c * CH, CH)], hb[b],
                        semg[b]).wait()
                @pl.when(c >= 2)
                def _():
                    pltpu.make_async_copy(
                        sg[b], acc.at[dsts[b]],
                        sems[b]).wait()

                @plsc.parallel_loop(0, CH, step=1, unroll=4)
                def _(r):
                    for g in range(D // 16):
                        s = pl.ds(g * 16, 16)
                        x = g1[b][r, s] + g2[b][r, s]
                        if has_b3:
                            x = x + hb[b][r, s]
                        hb[b][r, s] = x
                        sg[b][r, s] = _sigmoid16(x)
                _copy_idx(dstv[b], dsts[b])
                pltpu.async_copy(
                    hb[b], hat_h.at[pl.ds(tbase + c * CH, CH)], semw[b])
                pltpu.async_copy(
                    sg[b], acc.at[dsts[b]], sems[b],
                    add=True)
            return carry
        lax.fori_loop(0, NCH // 2, outer, 0)

        pltpu.make_async_copy(
            hb[1], hat_h.at[pl.ds(tbase + (NCH - 1) * CH, CH)], semw[1]).wait()
        pltpu.make_async_copy(
            sg[0], acc.at[dsts[0]], sems[0]).wait()
        pltpu.make_async_copy(
            sg[1], acc.at[dsts[1]], sems[1]).wait()

        plsc.subcore_barrier()
        pltpu.sync_copy(acc.at[pl.ds(base8, XRT)],
                        acc_h.at[cid, pl.ds(base8, XRT)])

    mesh = plsc.VectorSubcoreMesh(core_axis_name="c", subcore_axis_name="s",
                                  num_cores=NC, num_subcores=NS)
    f = pl.kernel(body, out_type=out_type, mesh=mesh,
                  scratch_types=tuple(scratch))
    args = [src, dst, b1, b2] + ([b3e] if has_b3 else [])
    return f(*args)


@jax.jit
def _sc_weighted(src, dst, hat, table):
    out_type = jax.ShapeDtypeStruct((NC, N, D), _f32)
    scratch = (
        [pltpu.VMEM((CH,), jnp.int32)] * 6
        + [pltpu.VMEM((CH, D), _f32)] * 6
        + [pltpu.VMEM_SHARED((N, D), _f32)]
        + [pltpu.SemaphoreType.DMA] * 4
    )

    def body(*refs):
        it = iter(refs)
        src_h = next(it); dst_h = next(it); hat_h = next(it); tab_h = next(it)
        acc_h = next(it)
        srcv = [next(it) for _ in range(2)]
        dstv = [next(it) for _ in range(2)]
        dsts = [next(it) for _ in range(2)]
        gt = [next(it) for _ in range(2)]
        hw = [next(it) for _ in range(2)]
        wo = [next(it) for _ in range(2)]
        acc = next(it)
        semg = [next(it) for _ in range(2)]
        sems = [next(it) for _ in range(2)]

        cid = lax.axis_index("c")
        sid = lax.axis_index("s")
        base8 = (sid * RPT) // 8 * 8
        tbase = (cid * NS + sid) * EPT

        _zero_acc(gt[0], acc, base8)
        plsc.subcore_barrier()

        def issue(c, b):
            base = tbase + c * CH
            pltpu.sync_copy(src_h.at[pl.ds(base, CH)], srcv[b])
            pltpu.sync_copy(dst_h.at[pl.ds(base, CH)], dstv[b])
            pltpu.async_copy(tab_h.at[srcv[b]], gt[b], semg[b])
            pltpu.async_copy(hat_h.at[pl.ds(base, CH)], hw[b], semg[b])

        issue(0, 0)

        def outer(cc, carry):
            for b in (0, 1):
                ob = 1 - b
                c = cc * 2 + b
                @pl.when(c + 1 < NCH)
                def _():
                    issue(c + 1, ob)
                pltpu.make_async_copy(tab_h.at[srcv[b]], gt[b], semg[b]).wait()
                pltpu.make_async_copy(
                    hat_h.at[pl.ds(tbase + c * CH, CH)], hw[b], semg[b]).wait()
                @pl.when(c >= 2)
                def _():
                    pltpu.make_async_copy(
                        wo[b], acc.at[dsts[b]], sems[b]).wait()

                @plsc.parallel_loop(0, CH, step=1, unroll=4)
                def _(r):
                    for g in range(D // 16):
                        s = pl.ds(g * 16, 16)
                        wo[b][r, s] = _sigmoid16(hw[b][r, s]) * gt[b][r, s]
                _copy_idx(dstv[b], dsts[b])
                pltpu.async_copy(wo[b], acc.at[dsts[b]], sems[b], add=True)
            return carry
        lax.fori_loop(0, NCH // 2, outer, 0)

        pltpu.make_async_copy(wo[0], acc.at[dsts[0]], sems[0]).wait()
        pltpu.make_async_copy(wo[1], acc.at[dsts[1]], sems[1]).wait()

        plsc.subcore_barrier()
        pltpu.sync_copy(acc.at[pl.ds(base8, XRT)],
                        acc_h.at[cid, pl.ds(base8, XRT)])

    mesh = plsc.VectorSubcoreMesh(core_axis_name="c", subcore_axis_name="s",
                                  num_cores=NC, num_subcores=NS)
    f = pl.kernel(body, out_type=out_type, mesh=mesh,
                  scratch_types=tuple(scratch))
    return f(src, dst, hat, table)


_BN_NODE = 1000
_BN_EDGE = 2000


def _dot(x, w):
    return lax.dot(x, w, preferred_element_type=_f32)


def _prep(h, p, wts, need_p):
    ws, bs = wts
    n_w, n_b = len(ws), len(bs)
    n_out = 6 if need_p else 4

    def body(*refs):
        it = iter(refs)
        h_ref = next(it); p_ref = next(it)
        w = [next(it) for _ in range(n_w)]
        b = [next(it) for _ in range(n_b)]
        outs = [next(it) for _ in range(n_out)]

        hf = h_ref[...]
        pf = p_ref[...]
        outs[0][...] = _dot(hf, w[0][...]) + b[0][...]
        outs[1][...] = _dot(hf, w[1][...]) + b[1][...]
        outs[2][...] = _dot(hf, w[2][...]) + _dot(pf, w[3][...]) + b[2][...]
        outs[3][...] = _dot(hf, w[4][...]) + _dot(pf, w[5][...]) + b[3][...]
        if need_p:
            outs[4][...] = _dot(pf, w[6][...]) + b[4][...]
            outs[5][...] = _dot(pf, w[7][...]) + b[5][...]

    grid = (N // _BN_NODE,)
    full = lambda shp: pl.BlockSpec(shp, lambda i: (0,) * len(shp))
    row_blk = pl.BlockSpec((_BN_NODE, D), lambda i: (i, 0))
    in_specs = [row_blk, row_blk] + [full((D, D))] * n_w + [full((1, D))] * n_b
    return pl.pallas_call(
        body, grid=grid, in_specs=in_specs,
        out_specs=[row_blk] * n_out,
        out_shape=[jax.ShapeDtypeStruct((N, D), _f32)] * n_out,
    )(h, p, *ws, *bs)


def _edge_linear(hat, w, b):
    def body(hat_ref, w_ref, b_ref, out_ref):
        e = jnp.maximum(hat_ref[...] * INV_BN, 0.0)
        out_ref[...] = _dot(e, w_ref[...]) + b_ref[...]

    grid = (E // _BN_EDGE,)
    return pl.pallas_call(
        body, grid=grid,
        in_specs=[
            pl.BlockSpec((_BN_EDGE, D), lambda i: (i, 0)),
            pl.BlockSpec((D, D), lambda i: (0, 0)),
            pl.BlockSpec((1, D), lambda i: (0, 0)),
        ],
        out_specs=pl.BlockSpec((_BN_EDGE, D), lambda i: (i, 0)),
        out_shape=jax.ShapeDtypeStruct((E, D), _f32),
    )(hat, w, b)


def _finish(a1, c1, asig, av, ap, need_p):
    def body(*refs):
        it = iter(refs)
        a1_ref = next(it)
        c1_ref = next(it) if need_p else None
        as_ref = next(it); av_ref = next(it)
        ap_ref = next(it) if need_p else None
        h_ref = next(it)
        p_ref = next(it) if need_p else None

        den = as_ref[0] + as_ref[1] + EPS
        sv = av_ref[0] + av_ref[1]
        hraw = a1_ref[...] + sv / den
        h_ref[...] = jnp.maximum(hraw * INV_BN, 0.0)
        if need_p:
            sp = ap_ref[0] + ap_ref[1]
            p_ref[...] = jnp.tanh(c1_ref[...] + sp / den)

    grid = (N // _BN_NODE,)
    row_blk = pl.BlockSpec((_BN_NODE, D), lambda i: (i, 0))
    acc_blk = pl.BlockSpec((NC, _BN_NODE, D), lambda i: (0, i, 0))
    n_out = 2 if need_p else 1
    in_specs = ([row_blk] + ([row_blk] if need_p else [])
                + [acc_blk] * (3 if need_p else 2))
    args = [a1] + ([c1] if need_p else []) + [asig, av] + ([ap] if need_p else [])
    return pl.pallas_call(
        body, grid=grid, in_specs=in_specs,
        out_specs=[row_blk] * n_out,
        out_shape=[jax.ShapeDtypeStruct((N, D), _f32)] * n_out,
    )(*args)


def _layer_weights(lp, extra_b, need_p):
    wa1 = lp["A1_w"].T
    wa2 = lp["A2_w"].T
    ws = [lp["B1_w"].T, lp["B2_w"].T, wa1[:D], wa1[D:], wa2[:D], wa2[D:]]
    bb2 = lp["B2_b"] + (extra_b if extra_b is not None else 0.0)
    bs = [lp["B1_b"], bb2, lp["A1_b"], lp["A2_b"]]
    if need_p:
        ws += [lp["C1_w"].T, lp["C2_w"].T]
        bs += [lp["C1_b"], lp["C2_b"]]
    return ([w.astype(_f32) for w in ws],
            [b.reshape(1, D).astype(_f32) for b in bs])


def kernel(features, edge_index, params):
    src = edge_index[0]
    dst = edge_index[1]
    h = features.astype(_f32)
    p = jnp.zeros_like(h)

    l1 = params["layer1"]
    b1, b2, a1, a2, c1, c2 = _prep(h, p, _layer_weights(l1, l1["B3_b"], True), True)
    hat, asig = _sc_hat_sig(src, dst, b1, b2, None, has_b3=False)
    av = _sc_weighted(src, dst, hat, a2)
    ap = _sc_weighted(src, dst, hat, c2)
    h, p = _finish(a1, c1, asig, av, ap, True)

    l2 = params["layer2"]
    b3e = _edge_linear(hat, l2["B3_w"].T.astype(_f32),
                       l2["B3_b"].reshape(1, D).astype(_f32))
    b1, b2, a1, a2, c1, c2 = _prep(h, p, _layer_weights(l2, None, True), True)
    hat, asig = _sc_hat_sig(src, dst, b1, b2, b3e, has_b3=True)
    av = _sc_weighted(src, dst, hat, a2)
    ap = _sc_weighted(src, dst, hat, c2)
    h, p = _finish(a1, c1, asig, av, ap, True)

    l3 = params["layer3"]
    b3e = _edge_linear(hat, l3["B3_w"].T.astype(_f32),
                       l3["B3_b"].reshape(1, D).astype(_f32))
    b1, b2, a1, a2 = _prep(h, p, _layer_weights(l3, None, False), False)
    hat, asig = _sc_hat_sig(src, dst, b1, b2, b3e, has_b3=True)
    av = _sc_weighted(src, dst, hat, a2)
    h = _finish(a1, None, asig, av, None, False)[0]
    return h


# speedup vs baseline: 1.5250x; 1.5250x over previous
import functools

import jax
import jax.numpy as jnp
from jax import lax
from jax.experimental import pallas as pl
from jax.experimental.pallas import tpu as pltpu
from jax.experimental.pallas import tpu_sc as plsc

N = 10000
E = 320000
D = 128
NS = 16
NC = 2
CH = 40
EPT = E // (NC * NS)
NCH = EPT // CH
RPT = N // NS
XRT = 632
INV_BN = float((1.0 + 1e-5) ** -0.5)
EPS = 1e-6

_f32 = jnp.float32


def _sigmoid16(x):
    return 1.0 / (1.0 + jnp.exp(-x))


def _zero_acc(buf, acc, base8):
    def zrow(r, carry):
        for g in range(D // 16):
            buf[r, pl.ds(g * 16, 16)] = jnp.zeros((16,), _f32)
        return carry
    lax.fori_loop(0, CH, zrow, 0)
    for q in range(XRT // CH):
        pltpu.sync_copy(buf, acc.at[pl.ds(base8 + q * CH, CH)])
    rem = XRT - (XRT // CH) * CH
    pltpu.sync_copy(buf.at[pl.ds(0, rem)],
                    acc.at[pl.ds(base8 + (XRT // CH) * CH, rem)])


def _copy_idx(src_ref, dst_ref):
    for off in (0, 16, CH - 16):
        s = pl.ds(off, 16)
        dst_ref[s] = src_ref[s]


@functools.partial(jax.jit, static_argnames=("has_b3",))
def _sc_hat_sig(src, dst, b1, b2, b3e, *, has_b3):
    out_type = (
        jax.ShapeDtypeStruct((E, D), _f32),
        jax.ShapeDtypeStruct((NC, N, D), _f32),
    )
    scratch = (
        [pltpu.VMEM((CH,), jnp.int32)] * 6
        + [pltpu.VMEM((CH, D), _f32)] * 8
        + [pltpu.VMEM_SHARED((N, D), _f32)]
        + [pltpu.SemaphoreType.DMA] * 8
    )

    def body(*refs):
        it = iter(refs)
        src_h = next(it); dst_h = next(it); b1_h = next(it); b2_h = next(it)
        b3e_h = next(it) if has_b3 else None
        hat_h = next(it); acc_h = next(it)
        srcv = [next(it) for _ in range(2)]
        dstv = [next(it) for _ in range(2)]
        dsts = [next(it) for _ in range(2)]
        g1 = [next(it) for _ in range(2)]
        g2 = [next(it) for _ in range(2)]
        hb = [next(it) for _ in range(2)]
        sg = [next(it) for _ in range(2)]
        acc = next(it)
        semg = [next(it) for _ in range(2)]
        semw = [next(it) for _ in range(2)]
        sems = [next(it) for _ in range(2)]
        semi = [next(it) for _ in range(2)]

        cid = lax.axis_index("c")
        sid = lax.axis_index("s")
        base8 = (sid * RPT) // 8 * 8
        tbase = (cid * NS + sid) * EPT

        _zero_acc(g1[0], acc, base8)
        plsc.subcore_barrier()

        def start_idx(c, b):
            base = tbase + c * CH
            pltpu.async_copy(src_h.at[pl.ds(base, CH)], srcv[b], semi[b])
            pltpu.async_copy(dst_h.at[pl.ds(base, CH)], dstv[b], semi[b])

        def wait_idx(c, b):
            base = tbase + c * CH
            pltpu.make_async_copy(src_h.at[pl.ds(base, CH)], srcv[b],
                                  semi[b]).wait()
            pltpu.make_async_copy(dst_h.at[pl.ds(base, CH)], dstv[b],
                                  semi[b]).wait()

        def issue(c, b):
            base = tbase + c * CH
            pltpu.async_copy(b1_h.at[srcv[b]], g1[b], semg[b])
            pltpu.async_copy(b2_h.at[dstv[b]], g2[b], semg[b])
            if has_b3:
                pltpu.async_copy(b3e_h.at[pl.ds(base, CH)], hb[b], semg[b])

        start_idx(0, 0)
        wait_idx(0, 0)
        issue(0, 0)

        def outer(cc, carry):
            for b in (0, 1):
                ob = 1 - b
                c = cc * 2 + b
                @pl.when(c + 1 < NCH)
                def _():
                    start_idx(c + 1, ob)
                @pl.when(c >= 1)
                def _():
                    pltpu.make_async_copy(
                        hb[ob], hat_h.at[pl.ds(tbase + (c - 1) * CH, CH)],
                        semw[ob]).wait()
                pltpu.make_async_copy(
                    b1_h.at[srcv[b]], g1[b], semg[b]).wait()
                pltpu.make_async_copy(
                    b2_h.at[dstv[b]], g2[b], semg[b]).wait()
                if has_b3:
                    pltpu.make_async_copy(
                        b3e_h.at[pl.ds(tbase + c * CH, CH)], hb[b],
                        semg[b]).wait()
                @pl.when(c + 1 < NCH)
                def _():
                    wait_idx(c + 1, ob)
                    issue(c + 1, ob)
                @pl.when(c >= 2)
                def _():
                    pltpu.make_async_copy(
                        sg[b], acc.at[dsts[b]],
                        sems[b]).wait()

                def row(r, rc):
                    for g in range(D // 16):
                        s = pl.ds(g * 16, 16)
                        x = g1[b][r, s] + g2[b][r, s]
                        if has_b3:
                            x = x + hb[b][r, s]
                        hb[b][r, s] = x
                        sg[b][r, s] = _sigmoid16(x)
                    return rc
                lax.fori_loop(0, CH, row, 0)
                _copy_idx(dstv[b], dsts[b])
                pltpu.async_copy(
                    hb[b], hat_h.at[pl.ds(tbase + c * CH, CH)], semw[b])
                pltpu.async_copy(
                    sg[b], acc.at[dsts[b]], sems[b],
                    add=True)
            return carry
        lax.fori_loop(0, NCH // 2, outer, 0)

        pltpu.make_async_copy(
            hb[1], hat_h.at[pl.ds(tbase + (NCH - 1) * CH, CH)], semw[1]).wait()
        pltpu.make_async_copy(
            sg[0], acc.at[dsts[0]], sems[0]).wait()
        pltpu.make_async_copy(
            sg[1], acc.at[dsts[1]], sems[1]).wait()

        plsc.subcore_barrier()
        pltpu.sync_copy(acc.at[pl.ds(base8, XRT)],
                        acc_h.at[cid, pl.ds(base8, XRT)])

    mesh = plsc.VectorSubcoreMesh(core_axis_name="c", subcore_axis_name="s",
                                  num_cores=NC, num_subcores=NS)
    f = pl.kernel(body, out_type=out_type, mesh=mesh,
                  scratch_types=tuple(scratch))
    args = [src, dst, b1, b2] + ([b3e] if has_b3 else [])
    return f(*args)


@jax.jit
def _sc_weighted(src, dst, hat, table):
    out_type = jax.ShapeDtypeStruct((NC, N, D), _f32)
    scratch = (
        [pltpu.VMEM((CH,), jnp.int32)] * 6
        + [pltpu.VMEM((CH, D), _f32)] * 6
        + [pltpu.VMEM_SHARED((N, D), _f32)]
        + [pltpu.SemaphoreType.DMA] * 6
    )

    def body(*refs):
        it = iter(refs)
        src_h = next(it); dst_h = next(it); hat_h = next(it); tab_h = next(it)
        acc_h = next(it)
        srcv = [next(it) for _ in range(2)]
        dstv = [next(it) for _ in range(2)]
        dsts = [next(it) for _ in range(2)]
        gt = [next(it) for _ in range(2)]
        hw = [next(it) for _ in range(2)]
        wo = [next(it) for _ in range(2)]
        acc = next(it)
        semg = [next(it) for _ in range(2)]
        sems = [next(it) for _ in range(2)]
        semi = [next(it) for _ in range(2)]

        cid = lax.axis_index("c")
        sid = lax.axis_index("s")
        base8 = (sid * RPT) // 8 * 8
        tbase = (cid * NS + sid) * EPT

        _zero_acc(gt[0], acc, base8)
        plsc.subcore_barrier()

        def start_idx(c, b):
            base = tbase + c * CH
            pltpu.async_copy(src_h.at[pl.ds(base, CH)], srcv[b], semi[b])
            pltpu.async_copy(dst_h.at[pl.ds(base, CH)], dstv[b], semi[b])

        def wait_idx(c, b):
            base = tbase + c * CH
            pltpu.make_async_copy(src_h.at[pl.ds(base, CH)], srcv[b],
                                  semi[b]).wait()
            pltpu.make_async_copy(dst_h.at[pl.ds(base, CH)], dstv[b],
                                  semi[b]).wait()

        def issue(c, b):
            base = tbase + c * CH
            pltpu.async_copy(tab_h.at[srcv[b]], gt[b], semg[b])
            pltpu.async_copy(hat_h.at[pl.ds(base, CH)], hw[b], semg[b])

        start_idx(0, 0)
        wait_idx(0, 0)
        issue(0, 0)

        def outer(cc, carry):
            for b in (0, 1):
                ob = 1 - b
                c = cc * 2 + b
                @pl.when(c + 1 < NCH)
                def _():
                    start_idx(c + 1, ob)
                pltpu.make_async_copy(tab_h.at[srcv[b]], gt[b], semg[b]).wait()
                pltpu.make_async_copy(
                    hat_h.at[pl.ds(tbase + c * CH, CH)], hw[b], semg[b]).wait()
                @pl.when(c + 1 < NCH)
                def _():
                    wait_idx(c + 1, ob)
                    issue(c + 1, ob)
                @pl.when(c >= 2)
                def _():
                    pltpu.make_async_copy(
                        wo[b], acc.at[dsts[b]], sems[b]).wait()

                def row(r, rc):
                    for g in range(D // 16):
                        s = pl.ds(g * 16, 16)
                        wo[b][r, s] = _sigmoid16(hw[b][r, s]) * gt[b][r, s]
                    return rc
                lax.fori_loop(0, CH, row, 0)
                _copy_idx(dstv[b], dsts[b])
                pltpu.async_copy(wo[b], acc.at[dsts[b]], sems[b], add=True)
            return carry
        lax.fori_loop(0, NCH // 2, outer, 0)

        pltpu.make_async_copy(wo[0], acc.at[dsts[0]], sems[0]).wait()
        pltpu.make_async_copy(wo[1], acc.at[dsts[1]], sems[1]).wait()

        plsc.subcore_barrier()
        pltpu.sync_copy(acc.at[pl.ds(base8, XRT)],
                        acc_h.at[cid, pl.ds(base8, XRT)])

    mesh = plsc.VectorSubcoreMesh(core_axis_name="c", subcore_axis_name="s",
                                  num_cores=NC, num_subcores=NS)
    f = pl.kernel(body, out_type=out_type, mesh=mesh,
                  scratch_types=tuple(scratch))
    return f(src, dst, hat, table)


_BN_NODE = 1000
_BN_EDGE = 2000


def _dot(x, w):
    return lax.dot(x, w, preferred_element_type=_f32)


def _prep(h, p, wts, need_p):
    ws, bs = wts
    n_w, n_b = len(ws), len(bs)
    n_out = 6 if need_p else 4

    def body(*refs):
        it = iter(refs)
        h_ref = next(it); p_ref = next(it)
        w = [next(it) for _ in range(n_w)]
        b = [next(it) for _ in range(n_b)]
        outs = [next(it) for _ in range(n_out)]

        hf = h_ref[...]
        pf = p_ref[...]
        outs[0][...] = _dot(hf, w[0][...]) + b[0][...]
        outs[1][...] = _dot(hf, w[1][...]) + b[1][...]
        outs[2][...] = _dot(hf, w[2][...]) + _dot(pf, w[3][...]) + b[2][...]
        outs[3][...] = _dot(hf, w[4][...]) + _dot(pf, w[5][...]) + b[3][...]
        if need_p:
            outs[4][...] = _dot(pf, w[6][...]) + b[4][...]
            outs[5][...] = _dot(pf, w[7][...]) + b[5][...]

    grid = (N // _BN_NODE,)
    full = lambda shp: pl.BlockSpec(shp, lambda i: (0,) * len(shp))
    row_blk = pl.BlockSpec((_BN_NODE, D), lambda i: (i, 0))
    in_specs = [row_blk, row_blk] + [full((D, D))] * n_w + [full((1, D))] * n_b
    return pl.pallas_call(
        body, grid=grid, in_specs=in_specs,
        out_specs=[row_blk] * n_out,
        out_shape=[jax.ShapeDtypeStruct((N, D), _f32)] * n_out,
    )(h, p, *ws, *bs)


def _edge_linear(hat, w, b):
    def body(hat_ref, w_ref, b_ref, out_ref):
        e = jnp.maximum(hat_ref[...] * INV_BN, 0.0)
        out_ref[...] = _dot(e, w_ref[...]) + b_ref[...]

    grid = (E // _BN_EDGE,)
    return pl.pallas_call(
        body, grid=grid,
        in_specs=[
            pl.BlockSpec((_BN_EDGE, D), lambda i: (i, 0)),
            pl.BlockSpec((D, D), lambda i: (0, 0)),
            pl.BlockSpec((1, D), lambda i: (0, 0)),
        ],
        out_specs=pl.BlockSpec((_BN_EDGE, D), lambda i: (i, 0)),
        out_shape=jax.ShapeDtypeStruct((E, D), _f32),
    )(hat, w, b)


def _finish(a1, c1, asig, av, ap, need_p):
    def body(*refs):
        it = iter(refs)
        a1_ref = next(it)
        c1_ref = next(it) if need_p else None
        as_ref = next(it); av_ref = next(it)
        ap_ref = next(it) if need_p else None
        h_ref = next(it)
        p_ref = next(it) if need_p else None

        den = as_ref[0] + as_ref[1] + EPS
        sv = av_ref[0] + av_ref[1]
        hraw = a1_ref[...] + sv / den
        h_ref[...] = jnp.maximum(hraw * INV_BN, 0.0)
        if need_p:
            sp = ap_ref[0] + ap_ref[1]
            p_ref[...] = jnp.tanh(c1_ref[...] + sp / den)

    grid = (N // _BN_NODE,)
    row_blk = pl.BlockSpec((_BN_NODE, D), lambda i: (i, 0))
    acc_blk = pl.BlockSpec((NC, _BN_NODE, D), lambda i: (0, i, 0))
    n_out = 2 if need_p else 1
    in_specs = ([row_blk] + ([row_blk] if need_p else [])
                + [acc_blk] * (3 if need_p else 2))
    args = [a1] + ([c1] if need_p else []) + [asig, av] + ([ap] if need_p else [])
    return pl.pallas_call(
        body, grid=grid, in_specs=in_specs,
        out_specs=[row_blk] * n_out,
        out_shape=[jax.ShapeDtypeStruct((N, D), _f32)] * n_out,
    )(*args)


def _layer_weights(lp, extra_b, need_p):
    wa1 = lp["A1_w"].T
    wa2 = lp["A2_w"].T
    ws = [lp["B1_w"].T, lp["B2_w"].T, wa1[:D], wa1[D:], wa2[:D], wa2[D:]]
    bb2 = lp["B2_b"] + (extra_b if extra_b is not None else 0.0)
    bs = [lp["B1_b"], bb2, lp["A1_b"], lp["A2_b"]]
    if need_p:
        ws += [lp["C1_w"].T, lp["C2_w"].T]
        bs += [lp["C1_b"], lp["C2_b"]]
    return ([w.astype(_f32) for w in ws],
            [b.reshape(1, D).astype(_f32) for b in bs])


def kernel(features, edge_index, params):
    src = edge_index[0]
    dst = edge_index[1]
    h = features.astype(_f32)
    p = jnp.zeros_like(h)

    l1 = params["layer1"]
    b1, b2, a1, a2, c1, c2 = _prep(h, p, _layer_weights(l1, l1["B3_b"], True), True)
    hat, asig = _sc_hat_sig(src, dst, b1, b2, None, has_b3=False)
    av = _sc_weighted(src, dst, hat, a2)
    ap = _sc_weighted(src, dst, hat, c2)
    h, p = _finish(a1, c1, asig, av, ap, True)

    l2 = params["layer2"]
    b3e = _edge_linear(hat, l2["B3_w"].T.astype(_f32),
                       l2["B3_b"].reshape(1, D).astype(_f32))
    b1, b2, a1, a2, c1, c2 = _prep(h, p, _layer_weights(l2, None, True), True)
    hat, asig = _sc_hat_sig(src, dst, b1, b2, b3e, has_b3=True)
    av = _sc_weighted(src, dst, hat, a2)
    ap = _sc_weighted(src, dst, hat, c2)
    h, p = _finish(a1, c1, asig, av, ap, True)

    l3 = params["layer3"]
    b3e = _edge_linear(hat, l3["B3_w"].T.astype(_f32),
                       l3["B3_b"].reshape(1, D).astype(_f32))
    b1, b2, a1, a2 = _prep(h, p, _layer_weights(l3, None, False), False)
    hat, asig = _sc_hat_sig(src, dst, b1, b2, b3e, has_b3=True)
    av = _sc_weighted(src, dst, hat, a2)
    h = _finish(a1, None, asig, av, None, False)[0]
    return h


# speedup vs baseline: 1.7181x; 1.1267x over previous
import functools

import jax
import jax.numpy as jnp
from jax import lax
from jax.experimental import pallas as pl
from jax.experimental.pallas import tpu as pltpu
from jax.experimental.pallas import tpu_sc as plsc

N = 10000
E = 320000
D = 128
NS = 16
NC = 2
CH = 40
EPT = E // (NC * NS)
NCH = EPT // CH
RPT = N // NS
XRT = 632
INV_BN = float((1.0 + 1e-5) ** -0.5)
EPS = 1e-6

_f32 = jnp.float32


def _sigmoid16(x):
    return 1.0 / (1.0 + jnp.exp(-x))


def _zero_acc(buf, acc, base8):
    def zrow(r, carry):
        for g in range(D // 16):
            buf[r, pl.ds(g * 16, 16)] = jnp.zeros((16,), _f32)
        return carry
    lax.fori_loop(0, CH, zrow, 0)
    for q in range(XRT // CH):
        pltpu.sync_copy(buf, acc.at[pl.ds(base8 + q * CH, CH)])
    rem = XRT - (XRT // CH) * CH
    pltpu.sync_copy(buf.at[pl.ds(0, rem)],
                    acc.at[pl.ds(base8 + (XRT // CH) * CH, rem)])


def _copy_idx(src_ref, dst_ref):
    for off in (0, 16, CH - 16):
        s = pl.ds(off, 16)
        dst_ref[s] = src_ref[s]


@functools.partial(jax.jit, static_argnames=("has_b3",))
def _sc_hat_sig(src, dst, b1, b2, b3e, *, has_b3):
    out_type = (
        jax.ShapeDtypeStruct((E, D), _f32),
        jax.ShapeDtypeStruct((NC, N, D), _f32),
    )
    scratch = (
        [pltpu.VMEM((CH,), jnp.int32)] * 6
        + [pltpu.VMEM((CH, D), _f32)] * 8
        + [pltpu.VMEM_SHARED((N, D), _f32)]
        + [pltpu.SemaphoreType.DMA] * 8
    )

    def body(*refs):
        it = iter(refs)
        src_h = next(it); dst_h = next(it); b1_h = next(it); b2_h = next(it)
        b3e_h = next(it) if has_b3 else None
        hat_h = next(it); acc_h = next(it)
        srcv = [next(it) for _ in range(2)]
        dstv = [next(it) for _ in range(2)]
        dsts = [next(it) for _ in range(2)]
        g1 = [next(it) for _ in range(2)]
        g2 = [next(it) for _ in range(2)]
        hb = [next(it) for _ in range(2)]
        sg = [next(it) for _ in range(2)]
        acc = next(it)
        semg = [next(it) for _ in range(2)]
        semw = [next(it) for _ in range(2)]
        sems = [next(it) for _ in range(2)]
        semi = [next(it) for _ in range(2)]

        cid = lax.axis_index("c")
        sid = lax.axis_index("s")
        base8 = (sid * RPT) // 8 * 8
        tbase = (cid * NS + sid) * EPT

        _zero_acc(g1[0], acc, base8)
        plsc.subcore_barrier()

        def start_idx(c, b):
            base = tbase + c * CH
            pltpu.async_copy(src_h.at[pl.ds(base, CH)], srcv[b], semi[b])
            pltpu.async_copy(dst_h.at[pl.ds(base, CH)], dstv[b], semi[b])

        def wait_idx(c, b):
            base = tbase + c * CH
            pltpu.make_async_copy(src_h.at[pl.ds(base, CH)], srcv[b],
                                  semi[b]).wait()
            pltpu.make_async_copy(dst_h.at[pl.ds(base, CH)], dstv[b],
                                  semi[b]).wait()

        def issue(c, b):
            base = tbase + c * CH
            pltpu.async_copy(b1_h.at[srcv[b]], g1[b], semg[b])
            pltpu.async_copy(b2_h.at[dstv[b]], g2[b], semg[b])
            if has_b3:
                pltpu.async_copy(b3e_h.at[pl.ds(base, CH)], hb[b], semg[b])

        start_idx(0, 0)
        wait_idx(0, 0)
        issue(0, 0)

        def outer(cc, carry):
            for b in (0, 1):
                ob = 1 - b
                c = cc * 2 + b
                @pl.when(c + 1 < NCH)
                def _():
                    start_idx(c + 1, ob)
                @pl.when(c >= 1)
                def _():
                    pltpu.make_async_copy(
                        hb[ob], hat_h.at[pl.ds(tbase + (c - 1) * CH, CH)],
                        semw[ob]).wait()
                pltpu.make_async_copy(
                    b1_h.at[srcv[b]], g1[b], semg[b]).wait()
                pltpu.make_async_copy(
                    b2_h.at[dstv[b]], g2[b], semg[b]).wait()
                if has_b3:
                    pltpu.make_async_copy(
                        b3e_h.at[pl.ds(tbase + c * CH, CH)], hb[b],
                        semg[b]).wait()
                @pl.when(c + 1 < NCH)
                def _():
                    wait_idx(c + 1, ob)
                    issue(c + 1, ob)
                @pl.when(c >= 2)
                def _():
                    pltpu.make_async_copy(
                        sg[b], acc.at[dsts[b]],
                        sems[b]).wait()

                def row(r, rc):
                    for g in range(D // 16):
                        s = pl.ds(g * 16, 16)
                        x = g1[b][r, s] + g2[b][r, s]
                        if has_b3:
                            x = x + hb[b][r, s]
                        hb[b][r, s] = x
                        sg[b][r, s] = _sigmoid16(x)
                    return rc
                lax.fori_loop(0, CH, row, 0)
                _copy_idx(dstv[b], dsts[b])
                pltpu.async_copy(
                    hb[b], hat_h.at[pl.ds(tbase + c * CH, CH)], semw[b])
                pltpu.async_copy(
                    sg[b], acc.at[dsts[b]], sems[b],
                    add=True)
            return carry
        lax.fori_loop(0, NCH // 2, outer, 0)

        pltpu.make_async_copy(
            hb[1], hat_h.at[pl.ds(tbase + (NCH - 1) * CH, CH)], semw[1]).wait()
        pltpu.make_async_copy(
            sg[0], acc.at[dsts[0]], sems[0]).wait()
        pltpu.make_async_copy(
            sg[1], acc.at[dsts[1]], sems[1]).wait()

        plsc.subcore_barrier()
        pltpu.sync_copy(acc.at[pl.ds(base8, XRT)],
                        acc_h.at[cid, pl.ds(base8, XRT)])

    mesh = plsc.VectorSubcoreMesh(core_axis_name="c", subcore_axis_name="s",
                                  num_cores=NC, num_subcores=NS)
    f = pl.kernel(body, out_type=out_type, mesh=mesh,
                  scratch_types=tuple(scratch))
    args = [src, dst, b1, b2] + ([b3e] if has_b3 else [])
    return f(*args)


@jax.jit
def _sc_weighted(src, dst, hat, table):
    out_type = jax.ShapeDtypeStruct((NC, N, D), _f32)
    scratch = (
        [pltpu.VMEM((CH,), jnp.int32)] * 6
        + [pltpu.VMEM((CH, D), _f32)] * 6
        + [pltpu.VMEM_SHARED((N, D), _f32)]
        + [pltpu.SemaphoreType.DMA] * 6
    )

    def body(*refs):
        it = iter(refs)
        src_h = next(it); dst_h = next(it); hat_h = next(it); tab_h = next(it)
        acc_h = next(it)
        srcv = [next(it) for _ in range(2)]
        dstv = [next(it) for _ in range(2)]
        dsts = [next(it) for _ in range(2)]
        gt = [next(it) for _ in range(2)]
        hw = [next(it) for _ in range(2)]
        wo = [next(it) for _ in range(2)]
        acc = next(it)
        semg = [next(it) for _ in range(2)]
        sems = [next(it) for _ in range(2)]
        semi = [next(it) for _ in range(2)]

        cid = lax.axis_index("c")
        sid = lax.axis_index("s")
        base8 = (sid * RPT) // 8 * 8
        tbase = (cid * NS + sid) * EPT

        _zero_acc(gt[0], acc, base8)
        plsc.subcore_barrier()

        def start_idx(c, b):
            base = tbase + c * CH
            pltpu.async_copy(src_h.at[pl.ds(base, CH)], srcv[b], semi[b])
            pltpu.async_copy(dst_h.at[pl.ds(base, CH)], dstv[b], semi[b])

        def wait_idx(c, b):
            base = tbase + c * CH
            pltpu.make_async_copy(src_h.at[pl.ds(base, CH)], srcv[b],
                                  semi[b]).wait()
            pltpu.make_async_copy(dst_h.at[pl.ds(base, CH)], dstv[b],
                                  semi[b]).wait()

        def issue(c, b):
            base = tbase + c * CH
            pltpu.async_copy(tab_h.at[srcv[b]], gt[b], semg[b])
            pltpu.async_copy(hat_h.at[pl.ds(base, CH)], hw[b], semg[b])

        start_idx(0, 0)
        wait_idx(0, 0)
        issue(0, 0)

        def outer(cc, carry):
            for b in (0, 1):
                ob = 1 - b
                c = cc * 2 + b
                @pl.when(c + 1 < NCH)
                def _():
                    start_idx(c + 1, ob)
                pltpu.make_async_copy(tab_h.at[srcv[b]], gt[b], semg[b]).wait()
                pltpu.make_async_copy(
                    hat_h.at[pl.ds(tbase + c * CH, CH)], hw[b], semg[b]).wait()
                @pl.when(c + 1 < NCH)
                def _():
                    wait_idx(c + 1, ob)
                    issue(c + 1, ob)
                @pl.when(c >= 2)
                def _():
                    pltpu.make_async_copy(
                        wo[b], acc.at[dsts[b]], sems[b]).wait()

                def row(r, rc):
                    for g in range(D // 16):
                        s = pl.ds(g * 16, 16)
                        wo[b][r, s] = _sigmoid16(hw[b][r, s]) * gt[b][r, s]
                    return rc
                lax.fori_loop(0, CH, row, 0)
                _copy_idx(dstv[b], dsts[b])
                pltpu.async_copy(wo[b], acc.at[dsts[b]], sems[b], add=True)
            return carry
        lax.fori_loop(0, NCH // 2, outer, 0)

        pltpu.make_async_copy(wo[0], acc.at[dsts[0]], sems[0]).wait()
        pltpu.make_async_copy(wo[1], acc.at[dsts[1]], sems[1]).wait()

        plsc.subcore_barrier()
        pltpu.sync_copy(acc.at[pl.ds(base8, XRT)],
                        acc_h.at[cid, pl.ds(base8, XRT)])

    mesh = plsc.VectorSubcoreMesh(core_axis_name="c", subcore_axis_name="s",
                                  num_cores=NC, num_subcores=NS)
    f = pl.kernel(body, out_type=out_type, mesh=mesh,
                  scratch_types=tuple(scratch))
    return f(src, dst, hat, table)


_BN_NODE = 1000
_BN_EDGE = 2000


def _dot(x, w):
    return lax.dot(x, w, preferred_element_type=_f32)


def _prep(h, p, wts, need_p):
    ws, bs = wts
    n_w, n_b = len(ws), len(bs)
    n_out = 6 if need_p else 4

    def body(*refs):
        it = iter(refs)
        h_ref = next(it); p_ref = next(it)
        w = [next(it) for _ in range(n_w)]
        b = [next(it) for _ in range(n_b)]
        outs = [next(it) for _ in range(n_out)]

        hf = h_ref[...]
        pf = p_ref[...]
        outs[0][...] = _dot(hf, w[0][...]) + b[0][...]
        outs[1][...] = _dot(hf, w[1][...]) + b[1][...]
        outs[2][...] = _dot(hf, w[2][...]) + _dot(pf, w[3][...]) + b[2][...]
        outs[3][...] = _dot(hf, w[4][...]) + _dot(pf, w[5][...]) + b[3][...]
        if need_p:
            outs[4][...] = _dot(pf, w[6][...]) + b[4][...]
            outs[5][...] = _dot(pf, w[7][...]) + b[5][...]

    grid = (N // _BN_NODE,)
    full = lambda shp: pl.BlockSpec(shp, lambda i: (0,) * len(shp))
    row_blk = pl.BlockSpec((_BN_NODE, D), lambda i: (i, 0))
    in_specs = [row_blk, row_blk] + [full((D, D))] * n_w + [full((1, D))] * n_b
    return pl.pallas_call(
        body, grid=grid, in_specs=in_specs,
        out_specs=[row_blk] * n_out,
        out_shape=[jax.ShapeDtypeStruct((N, D), _f32)] * n_out,
    )(h, p, *ws, *bs)


def _edge_linear(hat, w, b):
    def body(hat_ref, w_ref, b_ref, out_ref):
        e = jnp.maximum(hat_ref[...] * INV_BN, 0.0)
        out_ref[...] = _dot(e, w_ref[...]) + b_ref[...]

    grid = (E // _BN_EDGE,)
    return pl.pallas_call(
        body, grid=grid,
        in_specs=[
            pl.BlockSpec((_BN_EDGE, D), lambda i: (i, 0)),
            pl.BlockSpec((D, D), lambda i: (0, 0)),
            pl.BlockSpec((1, D), lambda i: (0, 0)),
        ],
        out_specs=pl.BlockSpec((_BN_EDGE, D), lambda i: (i, 0)),
        out_shape=jax.ShapeDtypeStruct((E, D), _f32),
    )(hat, w, b)


def _finish(a1, c1, asig, av, ap, need_p):
    def body(*refs):
        it = iter(refs)
        a1_ref = next(it)
        c1_ref = next(it) if need_p else None
        as_ref = next(it); av_ref = next(it)
        ap_ref = next(it) if need_p else None
        h_ref = next(it)
        p_ref = next(it) if need_p else None

        den = as_ref[0] + as_ref[1] + EPS
        sv = av_ref[0] + av_ref[1]
        hraw = a1_ref[...] + sv / den
        h_ref[...] = jnp.maximum(hraw * INV_BN, 0.0)
        if need_p:
            sp = ap_ref[0] + ap_ref[1]
            p_ref[...] = jnp.tanh(c1_ref[...] + sp / den)

    grid = (N // _BN_NODE,)
    row_blk = pl.BlockSpec((_BN_NODE, D), lambda i: (i, 0))
    acc_blk = pl.BlockSpec((NC, _BN_NODE, D), lambda i: (0, i, 0))
    n_out = 2 if need_p else 1
    in_specs = ([row_blk] + ([row_blk] if need_p else [])
                + [acc_blk] * (3 if need_p else 2))
    args = [a1] + ([c1] if need_p else []) + [asig, av] + ([ap] if need_p else [])
    return pl.pallas_call(
        body, grid=grid, in_specs=in_specs,
        out_specs=[row_blk] * n_out,
        out_shape=[jax.ShapeDtypeStruct((N, D), _f32)] * n_out,
    )(*args)


def _finish_l1(a1, asig, av, bc1, bc2):
    def body(a1_ref, as_ref, av_ref, bc1_ref, bc2_ref, h_ref, p_ref):
        s = as_ref[0] + as_ref[1]
        den = s + EPS
        sv = av_ref[0] + av_ref[1]
        hraw = a1_ref[...] + sv / den
        h_ref[...] = jnp.maximum(hraw * INV_BN, 0.0)
        p_ref[...] = jnp.tanh(bc1_ref[...] + bc2_ref[...] * (s / den))

    grid = (N // _BN_NODE,)
    row_blk = pl.BlockSpec((_BN_NODE, D), lambda i: (i, 0))
    acc_blk = pl.BlockSpec((NC, _BN_NODE, D), lambda i: (0, i, 0))
    bias_blk = pl.BlockSpec((1, D), lambda i: (0, 0))
    return pl.pallas_call(
        body, grid=grid,
        in_specs=[row_blk, acc_blk, acc_blk, bias_blk, bias_blk],
        out_specs=[row_blk, row_blk],
        out_shape=[jax.ShapeDtypeStruct((N, D), _f32)] * 2,
    )(a1, asig, av, bc1, bc2)


def _layer_weights(lp, extra_b, need_p):
    wa1 = lp["A1_w"].T
    wa2 = lp["A2_w"].T
    ws = [lp["B1_w"].T, lp["B2_w"].T, wa1[:D], wa1[D:], wa2[:D], wa2[D:]]
    bb2 = lp["B2_b"] + (extra_b if extra_b is not None else 0.0)
    bs = [lp["B1_b"], bb2, lp["A1_b"], lp["A2_b"]]
    if need_p:
        ws += [lp["C1_w"].T, lp["C2_w"].T]
        bs += [lp["C1_b"], lp["C2_b"]]
    return ([w.astype(_f32) for w in ws],
            [b.reshape(1, D).astype(_f32) for b in bs])


def kernel(features, edge_index, params):
    src = edge_index[0]
    dst = edge_index[1]
    h = features.astype(_f32)
    p = jnp.zeros_like(h)

    l1 = params["layer1"]
    b1, b2, a1, a2 = _prep(h, p, _layer_weights(l1, l1["B3_b"], False), False)
    hat, asig = _sc_hat_sig(src, dst, b1, b2, None, has_b3=False)
    av = _sc_weighted(src, dst, hat, a2)
    h, p = _finish_l1(a1, asig, av,
                      l1["C1_b"].reshape(1, D).astype(_f32),
                      l1["C2_b"].reshape(1, D).astype(_f32))

    l2 = params["layer2"]
    b3e = _edge_linear(hat, l2["B3_w"].T.astype(_f32),
                       l2["B3_b"].reshape(1, D).astype(_f32))
    b1, b2, a1, a2, c1, c2 = _prep(h, p, _layer_weights(l2, None, True), True)
    hat, asig = _sc_hat_sig(src, dst, b1, b2, b3e, has_b3=True)
    av = _sc_weighted(src, dst, hat, a2)
    ap = _sc_weighted(src, dst, hat, c2)
    h, p = _finish(a1, c1, asig, av, ap, True)

    l3 = params["layer3"]
    b3e = _edge_linear(hat, l3["B3_w"].T.astype(_f32),
                       l3["B3_b"].reshape(1, D).astype(_f32))
    b1, b2, a1, a2 = _prep(h, p, _layer_weights(l3, None, False), False)
    hat, asig = _sc_hat_sig(src, dst, b1, b2, b3e, has_b3=True)
    av = _sc_weighted(src, dst, hat, a2)
    h = _finish(a1, None, asig, av, None, False)[0]
    return h
